# baseline (speedup 1.0000x reference)
import functools

import jax
import jax.numpy as jnp
from jax.experimental import pallas as pl
from jax.experimental.pallas import tpu as pltpu
from jax.experimental.pallas import tpu_sc as plsc

_N = 10000
_E = 320000
_D = 128
_G = 256
_BN = 1000
_NBLK = _N // _BN

_NW = 32
_KPAIR = 160
_K0 = 112
_K1 = _KPAIR - _K0
_KHMAX = max(_K0, _K1) // 2
_PADROWS = 16 * _KPAIR
_KDEG = _PADROWS // _NW
_NACC = 10112
_TSLICE = _NACC // 16
_NDEG = 10240
_TSDEG = _NDEG // 16

_f32 = jnp.float32


def _sc_mesh():
  return plsc.VectorSubcoreMesh(
      core_axis_name="c", subcore_axis_name="s", num_cores=2, num_subcores=16)


def _deg_body(dst_hbm, zdeg_hbm, ones_hbm, deg_hbm, idx_v, ones_v, acc):
  c = jax.lax.axis_index("c")
  s = jax.lax.axis_index("s")
  wid = s * 2 + c
  sl = pl.ds(s * _TSDEG, _TSDEG)
  pltpu.sync_copy(zdeg_hbm, acc.at[sl])
  plsc.subcore_barrier()
  pltpu.sync_copy(dst_hbm.at[pl.ds(wid * _KDEG, _KDEG)], idx_v)
  pltpu.sync_copy(ones_hbm, ones_v)

  @pl.loop(0, _KDEG)
  def _(j):
    pltpu.sync_copy(ones_v, acc.at[idx_v.at[j]], add=True)

  plsc.subcore_barrier()
  pltpu.sync_copy(acc.at[sl], deg_hbm.at[pl.ds(c * _NDEG + s * _TSDEG,
                                               _TSDEG)])


def _scat_body(h_hbm, src_hbm, dst_hbm, zrow_hbm, out_hbm,
               srcv, dstv, rows, acc, gsem):
  c = jax.lax.axis_index("c")
  s = jax.lax.axis_index("s")
  sl = pl.ds(s * _TSLICE, _TSLICE)
  pltpu.sync_copy(zrow_hbm, acc.at[sl])
  plsc.subcore_barrier()
  rh = jnp.where(c == 0, _K0 // 2, _K1 // 2)
  for h in range(2):
    base = s * _KPAIR + c * _K0 + h * rh
    pltpu.sync_copy(src_hbm.at[pl.ds(base, _KHMAX)], srcv)
    pltpu.sync_copy(dst_hbm.at[pl.ds(base, _KHMAX)], dstv)
    pltpu.async_copy(h_hbm.at[srcv.at[0]], rows.at[0], gsem)

    @pl.loop(0, _KHMAX)
    def _(j):
      cur = jax.lax.rem(j, 2)

      @pl.when(j < rh)
      def _():
        pltpu.make_async_copy(h_hbm.at[srcv.at[j]], rows.at[cur], gsem).wait()

      @pl.when(j + 1 < rh)
      def _():
        pltpu.async_copy(h_hbm.at[srcv.at[j + 1]], rows.at[1 - cur], gsem)

      @pl.when(j < rh)
      def _():
        pltpu.sync_copy(rows.at[cur], acc.at[dstv.at[j]], add=True)

  plsc.subcore_barrier()
  pltpu.sync_copy(acc.at[sl], out_hbm.at[c, sl])


def _make_deg_kernel():
  return pl.kernel(
      _deg_body,
      out_type=jax.ShapeDtypeStruct((2 * _NDEG,), _f32),
      mesh=_sc_mesh(),
      scratch_types=[
          pltpu.VMEM((_KDEG, 128), jnp.int32),
          pltpu.VMEM((128,), _f32),
          pltpu.VMEM_SHARED((_NDEG,), _f32),
      ])


def _make_scat_kernel():
  return pl.kernel(
      _scat_body,
      out_type=jax.ShapeDtypeStruct((2, _NACC, _D), _f32),
      mesh=_sc_mesh(),
      scratch_types=[
          pltpu.VMEM((_KHMAX, 128), jnp.int32),
          pltpu.VMEM((_KHMAX, 128), jnp.int32),
          pltpu.VMEM((2, 128, _D), _f32),
          pltpu.VMEM_SHARED((_NACC, _D), _f32),
          pltpu.SemaphoreType.DMA,
      ])


def _mm0_body(x_ref, w_ref, da_ref, db_ref, hp_ref, dis_ref):
  deg = 1.0 + da_ref[...] + db_ref[...]
  dis = jax.lax.rsqrt(deg)
  h = jnp.dot(x_ref[...], w_ref[...], preferred_element_type=_f32)
  hp_ref[...] = h * dis
  dis_ref[...] = dis


def _bn_affine(s1, s2, g, be):
  mu = s1 * (1.0 / _N)
  var = s2 * (1.0 / _N) - mu * mu
  a = g * jax.lax.rsqrt(var + 1e-5)
  return a, be - mu * a


def _stats_body(sa_ref, sb_ref, hp_ref, dis_ref, b_ref, bat_ref, g_ref,
                be_ref, t_ref, s1_ref, s2_ref, pool_ref, cnt_ref, p_ref):
  i = pl.program_id(0)
  conv = dis_ref[...] * (sa_ref[...] + sb_ref[...] + hp_ref[...]) + b_ref[...]
  t = jnp.tanh(conv)
  t_ref[...] = t

  @pl.when(i == 0)
  def _():
    s1_ref[...] = jnp.zeros_like(s1_ref)
    s2_ref[...] = jnp.zeros_like(s2_ref)
    pool_ref[...] = jnp.zeros_like(pool_ref)
    cnt_ref[...] = jnp.zeros_like(cnt_ref)

  oh = (jax.lax.broadcasted_iota(jnp.int32, (_G, _BN), 0)
        == bat_ref[0]).astype(_f32)
  pool_ref[...] += jnp.dot(oh, t, preferred_element_type=_f32)
  cnt_ref[...] += jnp.sum(oh, axis=1, keepdims=True)
  s1_ref[...] += jnp.sum(t, axis=0, keepdims=True)
  s2_ref[...] += jnp.sum(t * t, axis=0, keepdims=True)

  @pl.when(i == _NBLK - 1)
  def _():
    a, cc = _bn_affine(s1_ref[...], s2_ref[...], g_ref[...], be_ref[...])
    cnt = cnt_ref[...]
    p_ref[...] = (pool_ref[...] * a + cnt * cc) / jnp.maximum(cnt, 1.0)


def _stats1_body(sa_ref, sb_ref, hp_ref, dis_ref, b_ref, bat_ref, cnt_ref,
                 g_ref, be_ref, p_ref, s1_ref, s2_ref, pool_ref):
  i = pl.program_id(0)
  conv = dis_ref[...] * (sa_ref[...] + sb_ref[...] + hp_ref[...]) + b_ref[...]
  t = jnp.tanh(conv)

  @pl.when(i == 0)
  def _():
    s1_ref[...] = jnp.zeros_like(s1_ref)
    s2_ref[...] = jnp.zeros_like(s2_ref)
    pool_ref[...] = jnp.zeros_like(pool_ref)

  oh = (jax.lax.broadcasted_iota(jnp.int32, (_G, _BN), 0)
        == bat_ref[0]).astype(_f32)
  pool_ref[...] += jnp.dot(oh, t, preferred_element_type=_f32)
  s1_ref[...] += jnp.sum(t, axis=0, keepdims=True)
  s2_ref[...] += jnp.sum(t * t, axis=0, keepdims=True)

  @pl.when(i == _NBLK - 1)
  def _():
    a, cc = _bn_affine(s1_ref[...], s2_ref[...], g_ref[...], be_ref[...])
    cnt = cnt_ref[...]
    p_ref[...] = (pool_ref[...] * a + cnt * cc) / jnp.maximum(cnt, 1.0)


def _mm1_body(t_ref, s1_ref, s2_ref, dis_ref, w_ref, g_ref, be_ref, hp_ref):
  a, cc = _bn_affine(s1_ref[...], s2_ref[...], g_ref[...], be_ref[...])
  y = t_ref[...] * a + cc
  h = jnp.dot(y, w_ref[...], preferred_element_type=_f32)
  hp_ref[...] = h * dis_ref[...]


def _row_spec(i_map=None):
  return pl.BlockSpec((_BN, _D), i_map or (lambda i: (i, 0)))


def _full_spec(shape):
  return pl.BlockSpec(shape, lambda i: (0,) * len(shape))


def _mm0(x, w0, da, db):
  return pl.pallas_call(
      _mm0_body,
      grid=(_NBLK,),
      in_specs=[_row_spec(), _full_spec((_D, _D)),
                pl.BlockSpec((_BN, 1), lambda i: (i, 0)),
                pl.BlockSpec((_BN, 1), lambda i: (i, 0))],
      out_specs=[_row_spec(), pl.BlockSpec((_BN, 1), lambda i: (i, 0))],
      out_shape=[jax.ShapeDtypeStruct((_N, _D), _f32),
                 jax.ShapeDtypeStruct((_N, 1), _f32)],
  )(x, w0, da, db)


def _stats0(sa, sb, hp, dis, b, bat, g, be):
  return pl.pallas_call(
      _stats_body,
      grid=(_NBLK,),
      in_specs=[_row_spec(), _row_spec(), _row_spec(),
                pl.BlockSpec((_BN, 1), lambda i: (i, 0)),
                _full_spec((1, _D)),
                pl.BlockSpec((1, 1, _BN), lambda i: (i, 0, 0)),
                _full_spec((1, _D)), _full_spec((1, _D))],
      out_specs=[_row_spec(), _full_spec((1, _D)), _full_spec((1, _D)),
                 _full_spec((_G, _D)), _full_spec((_G, 1)),
                 _full_spec((_G, _D))],
      out_shape=[jax.ShapeDtypeStruct((_N, _D), _f32),
                 jax.ShapeDtypeStruct((1, _D), _f32),
                 jax.ShapeDtypeStruct((1, _D), _f32),
                 jax.ShapeDtypeStruct((_G, _D), _f32),
                 jax.ShapeDtypeStruct((_G, 1), _f32),
                 jax.ShapeDtypeStruct((_G, _D), _f32)],
  )(sa, sb, hp, dis, b, bat, g, be)


def _stats1(sa, sb, hp, dis, b, bat, cnt, g, be):
  return pl.pallas_call(
      _stats1_body,
      grid=(_NBLK,),
      in_specs=[_row_spec(), _row_spec(), _row_spec(),
                pl.BlockSpec((_BN, 1), lambda i: (i, 0)),
                _full_spec((1, _D)),
                pl.BlockSpec((1, 1, _BN), lambda i: (i, 0, 0)),
                _full_spec((_G, 1)),
                _full_spec((1, _D)), _full_spec((1, _D))],
      out_specs=[_full_spec((_G, _D))],
      out_shape=[jax.ShapeDtypeStruct((_G, _D), _f32)],
      scratch_shapes=[pltpu.VMEM((1, _D), _f32),
                      pltpu.VMEM((1, _D), _f32),
                      pltpu.VMEM((_G, _D), _f32)],
  )(sa, sb, hp, dis, b, bat, cnt, g, be)[0]


def _mm1(t, s1, s2, dis, w1, g, be):
  return pl.pallas_call(
      _mm1_body,
      grid=(_NBLK,),
      in_specs=[_row_spec(), _full_spec((1, _D)), _full_spec((1, _D)),
                pl.BlockSpec((_BN, 1), lambda i: (i, 0)),
                _full_spec((_D, _D)), _full_spec((1, _D)),
                _full_spec((1, _D))],
      out_specs=[_row_spec()],
      out_shape=[jax.ShapeDtypeStruct((_N, _D), _f32)],
  )(t, s1, s2, dis, w1, g, be)[0]


@jax.jit
def kernel(x, W0, b0, g0, be0, W1, b1, g1, be1, edge_index, batch):
  src = edge_index[0]
  dst = edge_index[1]
  pad = _PADROWS * 128 - _E
  src_p = jnp.concatenate(
      [src, jnp.zeros((pad,), jnp.int32)]).reshape(_PADROWS, 128)
  dst_p = jnp.concatenate(
      [dst, jnp.full((pad,), _N, jnp.int32)]).reshape(_PADROWS, 128)

  zdeg = jnp.zeros((_TSDEG,), _f32)
  ones128 = jnp.ones((128,), _f32)
  zrow = jnp.zeros((_TSLICE, _D), _f32)

  degs = _make_deg_kernel()(dst_p, zdeg, ones128)
  da = degs[:_N].reshape(_N, 1)
  db = degs[_NDEG:_NDEG + _N].reshape(_N, 1)

  b0r = b0.reshape(1, _D)
  g0r = g0.reshape(1, _D)
  be0r = be0.reshape(1, _D)
  b1r = b1.reshape(1, _D)
  g1r = g1.reshape(1, _D)
  be1r = be1.reshape(1, _D)
  bat = batch.reshape(_NBLK, 1, _BN)

  scat = _make_scat_kernel()

  hp0, dis = _mm0(x, W0, da, db)
  s0 = scat(hp0, src_p, dst_p, zrow)
  t0, s1_0, s2_0, _, cnt, p0 = _stats0(
      s0[0, :_N], s0[1, :_N], hp0, dis, b0r, bat, g0r, be0r)

  hp1 = _mm1(t0, s1_0, s2_0, dis, W1, g0r, be0r)
  s1 = scat(hp1, src_p, dst_p, zrow)
  p1 = _stats1(s1[0, :_N], s1[1, :_N], hp1, dis, b1r, bat, cnt, g1r, be1r)

  return (p0, p1)


# speedup vs baseline: 1.0734x; 1.0734x over previous
import functools

import jax
import jax.numpy as jnp
from jax.experimental import pallas as pl
from jax.experimental.pallas import tpu as pltpu
from jax.experimental.pallas import tpu_sc as plsc

_N = 10000
_E = 320000
_D = 128
_G = 256
_BN = 1000
_NBLK = _N // _BN

_NW = 32
_KPAIR = 160
_K0 = 112
_K1 = _KPAIR - _K0
_KHMAX = max(_K0, _K1) // 2
_PADROWS = 16 * _KPAIR
_KDEG = _PADROWS // _NW
_NACC = 10112
_TSLICE = _NACC // 16
_NDEG = 10240
_TSDEG = _NDEG // 16

_f32 = jnp.float32


def _sc_mesh():
  return plsc.VectorSubcoreMesh(
      core_axis_name="c", subcore_axis_name="s", num_cores=2, num_subcores=16)


def _deg_body(dst_hbm, zdeg_hbm, ones_hbm, deg_hbm, idx_v, ones_v, acc):
  c = jax.lax.axis_index("c")
  s = jax.lax.axis_index("s")
  wid = s * 2 + c
  sl = pl.ds(s * _TSDEG, _TSDEG)
  pltpu.sync_copy(zdeg_hbm, acc.at[sl])
  plsc.subcore_barrier()
  pltpu.sync_copy(dst_hbm.at[pl.ds(wid * _KDEG, _KDEG)], idx_v)
  pltpu.sync_copy(ones_hbm, ones_v)

  @pl.loop(0, _KDEG)
  def _(j):
    pltpu.sync_copy(ones_v, acc.at[idx_v.at[j]], add=True)

  plsc.subcore_barrier()
  pltpu.sync_copy(acc.at[sl], deg_hbm.at[pl.ds(c * _NDEG + s * _TSDEG,
                                               _TSDEG)])


def _scat_body(h_hbm, src_hbm, dst_hbm, zrow_hbm, out_hbm,
               srcv, dstv, rows, acc, gsem):
  c = jax.lax.axis_index("c")
  s = jax.lax.axis_index("s")
  sl = pl.ds(s * _TSLICE, _TSLICE)
  pltpu.sync_copy(zrow_hbm, acc.at[sl])
  plsc.subcore_barrier()
  rh = jnp.where(c == 0, _K0 // 2, _K1 // 2)
  for h in range(2):
    base = s * _KPAIR + c * _K0 + h * rh
    pltpu.sync_copy(src_hbm.at[pl.ds(base, _KHMAX)], srcv)
    pltpu.sync_copy(dst_hbm.at[pl.ds(base, _KHMAX)], dstv)
    pltpu.async_copy(h_hbm.at[srcv.at[0]], rows.at[0], gsem)

    @pl.loop(0, _KHMAX)
    def _(j):
      cur = jax.lax.rem(j, 2)

      @pl.when(j < rh)
      def _():
        pltpu.make_async_copy(h_hbm.at[srcv.at[j]], rows.at[cur], gsem).wait()

      @pl.when(j + 1 < rh)
      def _():
        pltpu.async_copy(h_hbm.at[srcv.at[j + 1]], rows.at[1 - cur], gsem)

      @pl.when(j < rh)
      def _():
        pltpu.sync_copy(rows.at[cur], acc.at[dstv.at[j]], add=True)

  plsc.subcore_barrier()
  pltpu.sync_copy(acc.at[sl], out_hbm.at[c, sl])


def _make_deg_kernel():
  return pl.kernel(
      _deg_body,
      out_type=jax.ShapeDtypeStruct((2 * _NDEG,), _f32),
      mesh=_sc_mesh(),
      scratch_types=[
          pltpu.VMEM((_KDEG, 128), jnp.int32),
          pltpu.VMEM((128,), _f32),
          pltpu.VMEM_SHARED((_NDEG,), _f32),
      ])


def _make_scat_kernel():
  return pl.kernel(
      _scat_body,
      out_type=jax.ShapeDtypeStruct((2, _NACC, _D), _f32),
      mesh=_sc_mesh(),
      scratch_types=[
          pltpu.VMEM((_KHMAX, 128), jnp.int32),
          pltpu.VMEM((_KHMAX, 128), jnp.int32),
          pltpu.VMEM((2, 128, _D), _f32),
          pltpu.VMEM_SHARED((_NACC, _D), _f32),
          pltpu.SemaphoreType.DMA,
      ])


def _mm0_body(x_ref, w_ref, da_ref, db_ref, hp_ref, dis_ref):
  deg = 1.0 + da_ref[...] + db_ref[...]
  dis = jax.lax.rsqrt(deg)
  h = jnp.dot(x_ref[...], w_ref[...], preferred_element_type=_f32)
  hp_ref[...] = h * dis
  dis_ref[...] = dis


def _bn_affine(s1, s2, g, be):
  mu = s1 * (1.0 / _N)
  var = s2 * (1.0 / _N) - mu * mu
  a = g * jax.lax.rsqrt(var + 1e-5)
  return a, be - mu * a


def _stats_body(sa_ref, sb_ref, hp_ref, dis_ref, b_ref, bat_ref,
                t_ref, s1_ref, s2_ref, pool_ref, cnt_ref):
  i = pl.program_id(0)
  conv = dis_ref[...] * (sa_ref[...] + sb_ref[...] + hp_ref[...]) + b_ref[...]
  t = jnp.tanh(conv)
  t_ref[...] = t

  @pl.when(i == 0)
  def _():
    s1_ref[...] = jnp.zeros_like(s1_ref)
    s2_ref[...] = jnp.zeros_like(s2_ref)
    pool_ref[...] = jnp.zeros_like(pool_ref)
    cnt_ref[...] = jnp.zeros_like(cnt_ref)

  oh = (jax.lax.broadcasted_iota(jnp.int32, (_G, _BN), 0)
        == bat_ref[0]).astype(_f32)
  pool_ref[...] += jnp.dot(oh, t, preferred_element_type=_f32)
  cnt_ref[...] += jnp.sum(oh, axis=1, keepdims=True)
  s1_ref[...] += jnp.sum(t, axis=0, keepdims=True)
  s2_ref[...] += jnp.sum(t * t, axis=0, keepdims=True)


def _stats1_body(sa_ref, sb_ref, hp_ref, dis_ref, b_ref, bat_ref,
                 s1_ref, s2_ref, pool_ref):
  i = pl.program_id(0)
  conv = dis_ref[...] * (sa_ref[...] + sb_ref[...] + hp_ref[...]) + b_ref[...]
  t = jnp.tanh(conv)

  @pl.when(i == 0)
  def _():
    s1_ref[...] = jnp.zeros_like(s1_ref)
    s2_ref[...] = jnp.zeros_like(s2_ref)
    pool_ref[...] = jnp.zeros_like(pool_ref)

  oh = (jax.lax.broadcasted_iota(jnp.int32, (_G, _BN), 0)
        == bat_ref[0]).astype(_f32)
  pool_ref[...] += jnp.dot(oh, t, preferred_element_type=_f32)
  s1_ref[...] += jnp.sum(t, axis=0, keepdims=True)
  s2_ref[...] += jnp.sum(t * t, axis=0, keepdims=True)


def _mm1_body(t_ref, s1_ref, s2_ref, dis_ref, w_ref, g_ref, be_ref, hp_ref):
  a, cc = _bn_affine(s1_ref[...], s2_ref[...], g_ref[...], be_ref[...])
  y = t_ref[...] * a + cc
  h = jnp.dot(y, w_ref[...], preferred_element_type=_f32)
  hp_ref[...] = h * dis_ref[...]


def _row_spec(i_map=None):
  return pl.BlockSpec((_BN, _D), i_map or (lambda i: (i, 0)))


def _full_spec(shape):
  return pl.BlockSpec(shape, lambda i: (0,) * len(shape))


def _mm0(x, w0, da, db):
  return pl.pallas_call(
      _mm0_body,
      grid=(_NBLK,),
      in_specs=[_row_spec(), _full_spec((_D, _D)),
                pl.BlockSpec((_BN, 1), lambda i: (i, 0)),
                pl.BlockSpec((_BN, 1), lambda i: (i, 0))],
      out_specs=[_row_spec(), pl.BlockSpec((_BN, 1), lambda i: (i, 0))],
      out_shape=[jax.ShapeDtypeStruct((_N, _D), _f32),
                 jax.ShapeDtypeStruct((_N, 1), _f32)],
  )(x, w0, da, db)


def _stats0(sa, sb, hp, dis, b, bat):
  return pl.pallas_call(
      _stats_body,
      grid=(_NBLK,),
      in_specs=[_row_spec(), _row_spec(), _row_spec(),
                pl.BlockSpec((_BN, 1), lambda i: (i, 0)),
                _full_spec((1, _D)),
                pl.BlockSpec((1, 1, _BN), lambda i: (i, 0, 0))],
      out_specs=[_row_spec(), _full_spec((1, _D)), _full_spec((1, _D)),
                 _full_spec((_G, _D)), _full_spec((_G, 1))],
      out_shape=[jax.ShapeDtypeStruct((_N, _D), _f32),
                 jax.ShapeDtypeStruct((1, _D), _f32),
                 jax.ShapeDtypeStruct((1, _D), _f32),
                 jax.ShapeDtypeStruct((_G, _D), _f32),
                 jax.ShapeDtypeStruct((_G, 1), _f32)],
  )(sa, sb, hp, dis, b, bat)


def _stats1(sa, sb, hp, dis, b, bat):
  return pl.pallas_call(
      _stats1_body,
      grid=(_NBLK,),
      in_specs=[_row_spec(), _row_spec(), _row_spec(),
                pl.BlockSpec((_BN, 1), lambda i: (i, 0)),
                _full_spec((1, _D)),
                pl.BlockSpec((1, 1, _BN), lambda i: (i, 0, 0))],
      out_specs=[_full_spec((1, _D)), _full_spec((1, _D)),
                 _full_spec((_G, _D))],
      out_shape=[jax.ShapeDtypeStruct((1, _D), _f32),
                 jax.ShapeDtypeStruct((1, _D), _f32),
                 jax.ShapeDtypeStruct((_G, _D), _f32)],
  )(sa, sb, hp, dis, b, bat)


def _fin_body(s1_ref, s2_ref, pool_ref, cnt_ref, g_ref, be_ref, p_ref):
  a, cc = _bn_affine(s1_ref[...], s2_ref[...], g_ref[...], be_ref[...])
  cnt = cnt_ref[...]
  p_ref[...] = (pool_ref[...] * a + cnt * cc) / jnp.maximum(cnt, 1.0)


def _finalize(s1, s2, pool, cnt, g, be):
  return pl.pallas_call(
      _fin_body,
      out_shape=jax.ShapeDtypeStruct((_G, _D), _f32),
  )(s1, s2, pool, cnt, g, be)


def _mm1(t, s1, s2, dis, w1, g, be):
  return pl.pallas_call(
      _mm1_body,
      grid=(_NBLK,),
      in_specs=[_row_spec(), _full_spec((1, _D)), _full_spec((1, _D)),
                pl.BlockSpec((_BN, 1), lambda i: (i, 0)),
                _full_spec((_D, _D)), _full_spec((1, _D)),
                _full_spec((1, _D))],
      out_specs=[_row_spec()],
      out_shape=[jax.ShapeDtypeStruct((_N, _D), _f32)],
  )(t, s1, s2, dis, w1, g, be)[0]


@jax.jit
def kernel(x, W0, b0, g0, be0, W1, b1, g1, be1, edge_index, batch):
  src = edge_index[0]
  dst = edge_index[1]
  pad = _PADROWS * 128 - _E
  src_p = jnp.concatenate(
      [src, jnp.zeros((pad,), jnp.int32)]).reshape(_PADROWS, 128)
  dst_p = jnp.concatenate(
      [dst, jnp.full((pad,), _N, jnp.int32)]).reshape(_PADROWS, 128)

  zdeg = jnp.zeros((_TSDEG,), _f32)
  ones128 = jnp.ones((128,), _f32)
  zrow = jnp.zeros((_TSLICE, _D), _f32)

  degs = _make_deg_kernel()(dst_p, zdeg, ones128)
  da = degs[:_N].reshape(_N, 1)
  db = degs[_NDEG:_NDEG + _N].reshape(_N, 1)

  b0r = b0.reshape(1, _D)
  g0r = g0.reshape(1, _D)
  be0r = be0.reshape(1, _D)
  b1r = b1.reshape(1, _D)
  g1r = g1.reshape(1, _D)
  be1r = be1.reshape(1, _D)
  bat = batch.reshape(_NBLK, 1, _BN)

  scat = _make_scat_kernel()

  hp0, dis = _mm0(x, W0, da, db)
  s0 = scat(hp0, src_p, dst_p, zrow)
  t0, s1_0, s2_0, pool0, cnt = _stats0(
      s0[0, :_N], s0[1, :_N], hp0, dis, b0r, bat)
  p0 = _finalize(s1_0, s2_0, pool0, cnt, g0r, be0r)

  hp1 = _mm1(t0, s1_0, s2_0, dis, W1, g0r, be0r)
  s1 = scat(hp1, src_p, dst_p, zrow)
  s1_1, s2_1, pool1 = _stats1(s1[0, :_N], s1[1, :_N], hp1, dis, b1r, bat)
  p1 = _finalize(s1_1, s2_1, pool1, cnt, g1r, be1r)

  return (p0, p1)


# speedup vs baseline: 1.0827x; 1.0087x over previous
import functools

import jax
import jax.numpy as jnp
from jax.experimental import pallas as pl
from jax.experimental.pallas import tpu as pltpu
from jax.experimental.pallas import tpu_sc as plsc

_N = 10000
_E = 320000
_D = 128
_G = 256
_BN = 1000
_NBLK = _N // _BN

_NW = 32
_KPAIR = 160
_K0 = 112
_K1 = _KPAIR - _K0
_KHMAX = max(_K0, _K1) // 2
_PADROWS = 16 * _KPAIR
_KDEG = _PADROWS // _NW
_NACC = 10112
_TSLICE = _NACC // 16
_NDEG = 10240
_TSDEG = _NDEG // 16

_f32 = jnp.float32


def _sc_mesh():
  return plsc.VectorSubcoreMesh(
      core_axis_name="c", subcore_axis_name="s", num_cores=2, num_subcores=16)


def _deg_body(dst_hbm, zdeg_hbm, ones_hbm, deg_hbm, idx_v, ones_v, acc):
  c = jax.lax.axis_index("c")
  s = jax.lax.axis_index("s")
  wid = s * 2 + c
  sl = pl.ds(s * _TSDEG, _TSDEG)
  pltpu.sync_copy(zdeg_hbm, acc.at[sl])
  plsc.subcore_barrier()
  pltpu.sync_copy(dst_hbm.at[pl.ds(wid * _KDEG, _KDEG)], idx_v)
  pltpu.sync_copy(ones_hbm, ones_v)

  @pl.loop(0, _KDEG)
  def _(j):
    pltpu.sync_copy(ones_v, acc.at[idx_v.at[j]], add=True)

  plsc.subcore_barrier()
  pltpu.sync_copy(acc.at[sl], deg_hbm.at[pl.ds(c * _NDEG + s * _TSDEG,
                                               _TSDEG)])


def _scat_body(h_hbm, src_hbm, dst_hbm, zrow_hbm, out_hbm,
               srcv, dstv, rows, acc, gsem):
  c = jax.lax.axis_index("c")
  s = jax.lax.axis_index("s")
  sl = pl.ds(s * _TSLICE, _TSLICE)
  rh = jnp.where(c == 0, _K0 // 2, _K1 // 2)

  def _load_idx_and_prime(base):
    pltpu.sync_copy(src_hbm.at[pl.ds(base, _KHMAX)], srcv)
    pltpu.sync_copy(dst_hbm.at[pl.ds(base, _KHMAX)], dstv)
    pltpu.async_copy(h_hbm.at[srcv.at[0]], rows.at[0], gsem)

  base0 = s * _KPAIR + c * _K0
  _load_idx_and_prime(base0)
  pltpu.sync_copy(zrow_hbm, acc.at[sl])
  plsc.subcore_barrier()
  for h in range(2):
    if h == 1:
      _load_idx_and_prime(base0 + rh)

    @pl.loop(0, _KHMAX)
    def _(j):
      cur = jax.lax.rem(j, 2)

      @pl.when(j < rh)
      def _():
        pltpu.make_async_copy(h_hbm.at[srcv.at[j]], rows.at[cur], gsem).wait()

      @pl.when(j + 1 < rh)
      def _():
        pltpu.async_copy(h_hbm.at[srcv.at[j + 1]], rows.at[1 - cur], gsem)

      @pl.when(j < rh)
      def _():
        pltpu.sync_copy(rows.at[cur], acc.at[dstv.at[j]], add=True)

  plsc.subcore_barrier()
  pltpu.sync_copy(acc.at[sl], out_hbm.at[c, sl])


def _make_deg_kernel():
  return pl.kernel(
      _deg_body,
      out_type=jax.ShapeDtypeStruct((2 * _NDEG,), _f32),
      mesh=_sc_mesh(),
      scratch_types=[
          pltpu.VMEM((_KDEG, 128), jnp.int32),
          pltpu.VMEM((128,), _f32),
          pltpu.VMEM_SHARED((_NDEG,), _f32),
      ])


def _make_scat_kernel():
  return pl.kernel(
      _scat_body,
      out_type=jax.ShapeDtypeStruct((2, _NACC, _D), _f32),
      mesh=_sc_mesh(),
      scratch_types=[
          pltpu.VMEM((_KHMAX, 128), jnp.int32),
          pltpu.VMEM((_KHMAX, 128), jnp.int32),
          pltpu.VMEM((2, 128, _D), _f32),
          pltpu.VMEM_SHARED((_NACC, _D), _f32),
          pltpu.SemaphoreType.DMA,
      ])


def _mm0_body(x_ref, w_ref, da_ref, db_ref, hp_ref, dis_ref):
  deg = 1.0 + da_ref[...] + db_ref[...]
  dis = jax.lax.rsqrt(deg)
  h = jnp.dot(x_ref[...], w_ref[...], preferred_element_type=_f32)
  hp_ref[...] = h * dis
  dis_ref[...] = dis


def _bn_affine(s1, s2, g, be):
  mu = s1 * (1.0 / _N)
  var = s2 * (1.0 / _N) - mu * mu
  a = g * jax.lax.rsqrt(var + 1e-5)
  return a, be - mu * a


def _stats_body(sa_ref, sb_ref, hp_ref, dis_ref, b_ref, bat_ref,
                t_ref, s1_ref, s2_ref, pool_ref, cnt_ref):
  i = pl.program_id(0)
  conv = dis_ref[...] * (sa_ref[...] + sb_ref[...] + hp_ref[...]) + b_ref[...]
  t = jnp.tanh(conv)
  t_ref[...] = t

  @pl.when(i == 0)
  def _():
    s1_ref[...] = jnp.zeros_like(s1_ref)
    s2_ref[...] = jnp.zeros_like(s2_ref)
    pool_ref[...] = jnp.zeros_like(pool_ref)
    cnt_ref[...] = jnp.zeros_like(cnt_ref)

  oh = (jax.lax.broadcasted_iota(jnp.int32, (_G, _BN), 0)
        == bat_ref[0]).astype(_f32)
  pool_ref[...] += jnp.dot(oh, t, preferred_element_type=_f32)
  cnt_ref[...] += jnp.sum(oh, axis=1, keepdims=True)
  s1_ref[...] += jnp.sum(t, axis=0, keepdims=True)
  s2_ref[...] += jnp.sum(t * t, axis=0, keepdims=True)


def _stats1_body(sa_ref, sb_ref, hp_ref, dis_ref, b_ref, bat_ref,
                 s1_ref, s2_ref, pool_ref):
  i = pl.program_id(0)
  conv = dis_ref[...] * (sa_ref[...] + sb_ref[...] + hp_ref[...]) + b_ref[...]
  t = jnp.tanh(conv)

  @pl.when(i == 0)
  def _():
    s1_ref[...] = jnp.zeros_like(s1_ref)
    s2_ref[...] = jnp.zeros_like(s2_ref)
    pool_ref[...] = jnp.zeros_like(pool_ref)

  oh = (jax.lax.broadcasted_iota(jnp.int32, (_G, _BN), 0)
        == bat_ref[0]).astype(_f32)
  pool_ref[...] += jnp.dot(oh, t, preferred_element_type=_f32)
  s1_ref[...] += jnp.sum(t, axis=0, keepdims=True)
  s2_ref[...] += jnp.sum(t * t, axis=0, keepdims=True)


def _mm1_body(t_ref, s1_ref, s2_ref, dis_ref, w_ref, g_ref, be_ref, hp_ref):
  a, cc = _bn_affine(s1_ref[...], s2_ref[...], g_ref[...], be_ref[...])
  y = t_ref[...] * a + cc
  h = jnp.dot(y, w_ref[...], preferred_element_type=_f32)
  hp_ref[...] = h * dis_ref[...]


def _row_spec(i_map=None):
  return pl.BlockSpec((_BN, _D), i_map or (lambda i: (i, 0)))


def _full_spec(shape):
  return pl.BlockSpec(shape, lambda i: (0,) * len(shape))


def _mm0(x, w0, da, db):
  return pl.pallas_call(
      _mm0_body,
      grid=(_NBLK,),
      in_specs=[_row_spec(), _full_spec((_D, _D)),
                pl.BlockSpec((_BN, 1), lambda i: (i, 0)),
                pl.BlockSpec((_BN, 1), lambda i: (i, 0))],
      out_specs=[_row_spec(), pl.BlockSpec((_BN, 1), lambda i: (i, 0))],
      out_shape=[jax.ShapeDtypeStruct((_N, _D), _f32),
                 jax.ShapeDtypeStruct((_N, 1), _f32)],
  )(x, w0, da, db)


def _stats0(sa, sb, hp, dis, b, bat):
  return pl.pallas_call(
      _stats_body,
      grid=(_NBLK,),
      in_specs=[_row_spec(), _row_spec(), _row_spec(),
                pl.BlockSpec((_BN, 1), lambda i: (i, 0)),
                _full_spec((1, _D)),
                pl.BlockSpec((1, 1, _BN), lambda i: (i, 0, 0))],
      out_specs=[_row_spec(), _full_spec((1, _D)), _full_spec((1, _D)),
                 _full_spec((_G, _D)), _full_spec((_G, 1))],
      out_shape=[jax.ShapeDtypeStruct((_N, _D), _f32),
                 jax.ShapeDtypeStruct((1, _D), _f32),
                 jax.ShapeDtypeStruct((1, _D), _f32),
                 jax.ShapeDtypeStruct((_G, _D), _f32),
                 jax.ShapeDtypeStruct((_G, 1), _f32)],
  )(sa, sb, hp, dis, b, bat)


def _stats1(sa, sb, hp, dis, b, bat):
  return pl.pallas_call(
      _stats1_body,
      grid=(_NBLK,),
      in_specs=[_row_spec(), _row_spec(), _row_spec(),
                pl.BlockSpec((_BN, 1), lambda i: (i, 0)),
                _full_spec((1, _D)),
                pl.BlockSpec((1, 1, _BN), lambda i: (i, 0, 0))],
      out_specs=[_full_spec((1, _D)), _full_spec((1, _D)),
                 _full_spec((_G, _D))],
      out_shape=[jax.ShapeDtypeStruct((1, _D), _f32),
                 jax.ShapeDtypeStruct((1, _D), _f32),
                 jax.ShapeDtypeStruct((_G, _D), _f32)],
  )(sa, sb, hp, dis, b, bat)


def _fin_body(s1_ref, s2_ref, pool_ref, cnt_ref, g_ref, be_ref, p_ref):
  a, cc = _bn_affine(s1_ref[...], s2_ref[...], g_ref[...], be_ref[...])
  cnt = cnt_ref[...]
  p_ref[...] = (pool_ref[...] * a + cnt * cc) / jnp.maximum(cnt, 1.0)


def _finalize(s1, s2, pool, cnt, g, be):
  return pl.pallas_call(
      _fin_body,
      out_shape=jax.ShapeDtypeStruct((_G, _D), _f32),
  )(s1, s2, pool, cnt, g, be)


def _mm1(t, s1, s2, dis, w1, g, be):
  return pl.pallas_call(
      _mm1_body,
      grid=(_NBLK,),
      in_specs=[_row_spec(), _full_spec((1, _D)), _full_spec((1, _D)),
                pl.BlockSpec((_BN, 1), lambda i: (i, 0)),
                _full_spec((_D, _D)), _full_spec((1, _D)),
                _full_spec((1, _D))],
      out_specs=[_row_spec()],
      out_shape=[jax.ShapeDtypeStruct((_N, _D), _f32)],
  )(t, s1, s2, dis, w1, g, be)[0]


@jax.jit
def kernel(x, W0, b0, g0, be0, W1, b1, g1, be1, edge_index, batch):
  src = edge_index[0]
  dst = edge_index[1]
  pad = _PADROWS * 128 - _E
  src_p = jnp.concatenate(
      [src, jnp.zeros((pad,), jnp.int32)]).reshape(_PADROWS, 128)
  dst_p = jnp.concatenate(
      [dst, jnp.full((pad,), _N, jnp.int32)]).reshape(_PADROWS, 128)

  zdeg = jnp.zeros((_TSDEG,), _f32)
  ones128 = jnp.ones((128,), _f32)
  zrow = jnp.zeros((_TSLICE, _D), _f32)

  degs = _make_deg_kernel()(dst_p, zdeg, ones128)
  da = degs[:_N].reshape(_N, 1)
  db = degs[_NDEG:_NDEG + _N].reshape(_N, 1)

  b0r = b0.reshape(1, _D)
  g0r = g0.reshape(1, _D)
  be0r = be0.reshape(1, _D)
  b1r = b1.reshape(1, _D)
  g1r = g1.reshape(1, _D)
  be1r = be1.reshape(1, _D)
  bat = batch.reshape(_NBLK, 1, _BN)

  scat = _make_scat_kernel()

  hp0, dis = _mm0(x, W0, da, db)
  s0 = scat(hp0, src_p, dst_p, zrow)
  t0, s1_0, s2_0, pool0, cnt = _stats0(
      s0[0, :_N], s0[1, :_N], hp0, dis, b0r, bat)
  p0 = _finalize(s1_0, s2_0, pool0, cnt, g0r, be0r)

  hp1 = _mm1(t0, s1_0, s2_0, dis, W1, g0r, be0r)
  s1 = scat(hp1, src_p, dst_p, zrow)
  s1_1, s2_1, pool1 = _stats1(s1[0, :_N], s1[1, :_N], hp1, dis, b1r, bat)
  p1 = _finalize(s1_1, s2_1, pool1, cnt, g1r, be1r)

  return (p0, p1)
